# baseline (speedup 1.0000x reference)
import functools

import jax
import jax.numpy as jnp
from jax import lax
from jax.experimental import pallas as pl
from jax.experimental.pallas import tpu as pltpu
from jax.experimental.pallas import tpu_sc as plsc

NUM_S = 4194304
NUM_R = 65536
NC = 2
NS = 16
NW = NC * NS
LANE = 128
ROWS = NUM_S // LANE
ROWS_PER_W = ROWS // NW
CHUNK_ROWS = 128
N_CHUNKS = ROWS_PER_W // CHUNK_ROWS

PREP_BLK = 1024
PREP_GRID = ROWS // PREP_BLK


def _prep_body(w_ref, s_ref, e_ref, val_ref, mn_ref, mx_ref):
    steps = (s_ref[...] + e_ref[...]) * 0.5
    val_ref[...] = w_ref[...] * steps
    mn_ref[...] = jnp.full((1, 1, LANE), jnp.min(steps), jnp.float32)
    mx_ref[...] = jnp.full((1, 1, LANE), jnp.max(steps), jnp.float32)


def _tc_prep(w, s, e):
    blk = pl.BlockSpec((PREP_BLK, LANE), lambda g: (g, 0))
    row = pl.BlockSpec((1, 1, LANE), lambda g: (g, 0, 0))
    return pl.pallas_call(
        _prep_body,
        grid=(PREP_GRID,),
        in_specs=[blk, blk, blk],
        out_specs=[blk, row, row],
        out_shape=[
            jax.ShapeDtypeStruct((ROWS, LANE), jnp.float32),
            jax.ShapeDtypeStruct((PREP_GRID, 1, LANE), jnp.float32),
            jax.ShapeDtypeStruct((PREP_GRID, 1, LANE), jnp.float32),
        ],
    )(w, s, e)


SAMP_PER_W = NUM_S // NW
SAMP_PER_CHUNK = CHUNK_ROWS * LANE
WIN = 16384
FLUSH = 2048


def _sc_body(val_hbm, w_hbm, idx_hbm, tabs_hbm,
             valb, wb, idxb, wdtab, watab, fidxb, zb, dtab, atab, sem):
    c = lax.axis_index("c")
    s = lax.axis_index("s")
    wid = c * NS + s

    stripe = NUM_R // NS

    def _zero(i, _):
        zb[pl.ds(i * 16, 16)] = jnp.zeros((16,), jnp.float32)
        return 0

    lax.fori_loop(0, stripe // 16, _zero, 0)
    pltpu.sync_copy(zb, dtab.at[pl.ds(s * stripe, stripe)])
    pltpu.sync_copy(zb, atab.at[pl.ds(s * stripe, stripe)])

    def _zl(i, _):
        z = jnp.zeros((16,), jnp.float32)
        wdtab[pl.ds(i * 16, 16)] = z
        watab[pl.ds(i * 16, 16)] = z
        return 0

    lax.fori_loop(0, WIN // 16, _zl, 0)
    plsc.subcore_barrier()

    s0 = wid * SAMP_PER_W

    def _chunk(ck, carry):
        f, maxs = carry
        sb = s0 + ck * SAMP_PER_CHUNK
        pltpu.sync_copy(val_hbm.at[pl.ds(sb, SAMP_PER_CHUNK)], valb)
        pltpu.sync_copy(w_hbm.at[pl.ds(sb, SAMP_PER_CHUNK)], wb)
        pltpu.sync_copy(idx_hbm.at[pl.ds(sb, SAMP_PER_CHUNK)], idxb)
        f = lax.select(ck == 0, idxb[pl.ds(0, 16)][0], f)
        span = idxb[pl.ds(SAMP_PER_CHUNK - 16, 16)][15] - f + 1
        fast = span <= WIN

        def _fast():
            fv = jnp.full((16,), f, jnp.int32)

            def _vec(j, _):
                base = j * 16
                iv = idxb[pl.ds(base, 16)] - fv
                plsc.addupdate_scatter(wdtab, [iv], valb[pl.ds(base, 16)])
                plsc.addupdate_scatter(watab, [iv], wb[pl.ds(base, 16)])
                return 0

            lax.fori_loop(0, SAMP_PER_CHUNK // 16, _vec, 0)

        def _slow():
            d_cp = pltpu.async_copy(valb, dtab.at[idxb], sem, add=True)
            a_cp = pltpu.async_copy(wb, atab.at[idxb], sem, add=True)
            d_cp.wait()
            a_cp.wait()

        lax.cond(fast, _fast, _slow)
        maxs = lax.select(fast, jnp.maximum(maxs, span), maxs)
        return f, maxs

    f, maxs = lax.fori_loop(0, N_CHUNKS, _chunk,
                            (jnp.int32(0), jnp.int32(0)))

    nblk = (maxs + FLUSH - 1) // FLUSH
    lane16 = lax.iota(jnp.int32, 16)

    def _flush(b, _):
        base = f + b * FLUSH

        def _bld(j, _):
            fidxb[pl.ds(j * 16, 16)] = jnp.minimum(
                lane16 + (base + j * 16), NUM_R - 1)
            return 0

        lax.fori_loop(0, FLUSH // 16, _bld, 0)
        off = b * FLUSH
        d_cp = pltpu.async_copy(wdtab.at[pl.ds(off, FLUSH)],
                                dtab.at[fidxb], sem, add=True)
        a_cp = pltpu.async_copy(watab.at[pl.ds(off, FLUSH)],
                                atab.at[fidxb], sem, add=True)
        d_cp.wait()
        a_cp.wait()
        return 0

    lax.fori_loop(0, nblk, _flush, 0)
    plsc.subcore_barrier()

    @pl.when(s == 0)
    def _dump():
        pltpu.sync_copy(dtab, tabs_hbm.at[c, 0])
        pltpu.sync_copy(atab, tabs_hbm.at[c, 1])


def _sc_scatter(val, w, idx):
    mesh = plsc.VectorSubcoreMesh(core_axis_name="c", subcore_axis_name="s")
    return pl.kernel(
        _sc_body,
        out_type=jax.ShapeDtypeStruct((NC, 2, NUM_R), jnp.float32),
        mesh=mesh,
        scratch_types=[
            pltpu.VMEM((CHUNK_ROWS * LANE,), jnp.float32),
            pltpu.VMEM((CHUNK_ROWS * LANE,), jnp.float32),
            pltpu.VMEM((CHUNK_ROWS * LANE,), jnp.int32),
            pltpu.VMEM((WIN,), jnp.float32),
            pltpu.VMEM((WIN,), jnp.float32),
            pltpu.VMEM((FLUSH,), jnp.int32),
            pltpu.VMEM((NUM_R // NS,), jnp.float32),
            pltpu.VMEM_SHARED((NUM_R,), jnp.float32),
            pltpu.VMEM_SHARED((NUM_R,), jnp.float32),
            pltpu.SemaphoreType.DMA,
        ],
        compiler_params=pltpu.CompilerParams(needs_layout_passes=False),
    )(val, w, idx)


def _finish_body(d0_ref, d1_ref, a0_ref, a1_ref, mn_ref, mx_ref, fac_ref,
                 out_ref):
    depth = (d0_ref[...] + d1_ref[...]) / (a0_ref[...] + a1_ref[...] + 1e-10)
    mn = jnp.min(mn_ref[...])
    mx = jnp.max(mx_ref[...])
    out_ref[...] = jnp.clip(depth, mn, mx) * fac_ref[...]


def _tc_finish(d0, d1, a0, a1, mn, mx, fac):
    return pl.pallas_call(
        _finish_body,
        out_shape=jax.ShapeDtypeStruct((NUM_R // LANE, LANE), jnp.float32),
    )(d0, d1, a0, a1, mn, mx, fac)


def kernel(weights, starts, ends, factor_depth_coords, ray_indices, num_rays):
    del num_rays
    w2 = weights.reshape(ROWS, LANE)
    s2 = starts.reshape(ROWS, LANE)
    e2 = ends.reshape(ROWS, LANE)
    val, mn, mx = _tc_prep(w2, s2, e2)
    idx = ray_indices.astype(jnp.int32)
    tabs = _sc_scatter(val.reshape(NUM_S), weights.reshape(NUM_S), idx)
    tr = NUM_R // LANE
    d0 = tabs[0, 0].reshape(tr, LANE)
    d1 = tabs[1, 0].reshape(tr, LANE)
    a0 = tabs[0, 1].reshape(tr, LANE)
    a1 = tabs[1, 1].reshape(tr, LANE)
    fac = factor_depth_coords.reshape(tr, LANE)
    out = _tc_finish(d0, d1, a0, a1, mn, mx, fac)
    return out.reshape(NUM_R, 1)


# speedup vs baseline: 1.0007x; 1.0007x over previous
import jax
import jax.numpy as jnp
from jax import lax
from jax.experimental import pallas as pl
from jax.experimental.pallas import tpu as pltpu
from jax.experimental.pallas import tpu_sc as plsc

NUM_S = 4194304
NUM_R = 65536
NC = 2
NS = 16
NW = NC * NS
LANE = 128
ROWS = NUM_S // LANE

PREP_BLK = 1024
PREP_GRID = ROWS // PREP_BLK

SAMP_PER_W = NUM_S // NW
SAMP_PER_CHUNK = 16384
N_CHUNKS = SAMP_PER_W // SAMP_PER_CHUNK
WIN = 16384
FLUSH = 2048
UNROLL = 8


def _prep_body(w_ref, s_ref, e_ref, val_ref, mn_ref, mx_ref):
    steps = (s_ref[...] + e_ref[...]) * 0.5
    val_ref[...] = w_ref[...] * steps
    mn_ref[...] = jnp.full((1, 1, LANE), jnp.min(steps), jnp.float32)
    mx_ref[...] = jnp.full((1, 1, LANE), jnp.max(steps), jnp.float32)


def _tc_prep(w, s, e):
    blk = pl.BlockSpec((PREP_BLK, LANE), lambda g: (g, 0))
    row = pl.BlockSpec((1, 1, LANE), lambda g: (g, 0, 0))
    return pl.pallas_call(
        _prep_body,
        grid=(PREP_GRID,),
        in_specs=[blk, blk, blk],
        out_specs=[blk, row, row],
        out_shape=[
            jax.ShapeDtypeStruct((ROWS, LANE), jnp.float32),
            jax.ShapeDtypeStruct((PREP_GRID, 1, LANE), jnp.float32),
            jax.ShapeDtypeStruct((PREP_GRID, 1, LANE), jnp.float32),
        ],
    )(w, s, e)


def _sc_body(val_hbm, w_hbm, idx_hbm, tabs_hbm,
             valb, wb, idxb, wdtab, watab, fidxb, zb, dtab, atab, sem):
    c = lax.axis_index("c")
    s = lax.axis_index("s")
    wid = c * NS + s

    stripe = NUM_R // NS

    def _zero(i, _):
        zb[pl.ds(i * 16, 16)] = jnp.zeros((16,), jnp.float32)
        return 0

    lax.fori_loop(0, stripe // 16, _zero, 0)
    pltpu.sync_copy(zb, dtab.at[pl.ds(s * stripe, stripe)])
    pltpu.sync_copy(zb, atab.at[pl.ds(s * stripe, stripe)])

    def _zl(i, _):
        z = jnp.zeros((16,), jnp.float32)
        wdtab[pl.ds(i * 16, 16)] = z
        watab[pl.ds(i * 16, 16)] = z
        return 0

    lax.fori_loop(0, WIN // 16, _zl, 0)
    plsc.subcore_barrier()

    s0 = wid * SAMP_PER_W

    def _chunk(ck, carry):
        f, maxs = carry
        sb = s0 + ck * SAMP_PER_CHUNK
        pltpu.sync_copy(val_hbm.at[pl.ds(sb, SAMP_PER_CHUNK)], valb)
        pltpu.sync_copy(w_hbm.at[pl.ds(sb, SAMP_PER_CHUNK)], wb)
        pltpu.sync_copy(idx_hbm.at[pl.ds(sb, SAMP_PER_CHUNK)], idxb)
        f = lax.select(ck == 0, idxb[pl.ds(0, 16)][0], f)
        span = idxb[pl.ds(SAMP_PER_CHUNK - 16, 16)][15] - f + 1
        fast = span <= WIN

        def _fast():
            fv = jnp.full((16,), f, jnp.int32)

            def _vec(j, _):
                for k in range(UNROLL):
                    base = (j * UNROLL + k) * 16
                    iv = idxb[pl.ds(base, 16)] - fv
                    plsc.addupdate_scatter(wdtab, [iv],
                                           valb[pl.ds(base, 16)])
                    plsc.addupdate_scatter(watab, [iv],
                                           wb[pl.ds(base, 16)])
                return 0

            lax.fori_loop(0, SAMP_PER_CHUNK // (16 * UNROLL), _vec, 0)

        def _slow():
            d_cp = pltpu.async_copy(valb, dtab.at[idxb], sem, add=True)
            a_cp = pltpu.async_copy(wb, atab.at[idxb], sem, add=True)
            d_cp.wait()
            a_cp.wait()

        lax.cond(fast, _fast, _slow)
        maxs = lax.select(fast, jnp.maximum(maxs, span), maxs)
        return f, maxs

    f, maxs = lax.fori_loop(0, N_CHUNKS, _chunk,
                            (jnp.int32(0), jnp.int32(0)))

    nblk = (maxs + FLUSH - 1) // FLUSH
    lane16 = lax.iota(jnp.int32, 16)

    def _flush(b, _):
        base = f + b * FLUSH

        def _bld(j, _):
            fidxb[pl.ds(j * 16, 16)] = jnp.minimum(
                lane16 + (base + j * 16), NUM_R - 1)
            return 0

        lax.fori_loop(0, FLUSH // 16, _bld, 0)
        off = b * FLUSH
        d_cp = pltpu.async_copy(wdtab.at[pl.ds(off, FLUSH)],
                                dtab.at[fidxb], sem, add=True)
        a_cp = pltpu.async_copy(watab.at[pl.ds(off, FLUSH)],
                                atab.at[fidxb], sem, add=True)
        d_cp.wait()
        a_cp.wait()
        return 0

    lax.fori_loop(0, nblk, _flush, 0)
    plsc.subcore_barrier()

    @pl.when(s == 0)
    def _dump():
        pltpu.sync_copy(dtab, tabs_hbm.at[c, 0])
        pltpu.sync_copy(atab, tabs_hbm.at[c, 1])


def _sc_scatter(val, w, idx):
    mesh = plsc.VectorSubcoreMesh(core_axis_name="c", subcore_axis_name="s")
    return pl.kernel(
        _sc_body,
        out_type=jax.ShapeDtypeStruct((NC, 2, NUM_R), jnp.float32),
        mesh=mesh,
        scratch_types=[
            pltpu.VMEM((SAMP_PER_CHUNK,), jnp.float32),
            pltpu.VMEM((SAMP_PER_CHUNK,), jnp.float32),
            pltpu.VMEM((SAMP_PER_CHUNK,), jnp.int32),
            pltpu.VMEM((WIN,), jnp.float32),
            pltpu.VMEM((WIN,), jnp.float32),
            pltpu.VMEM((FLUSH,), jnp.int32),
            pltpu.VMEM((NUM_R // NS,), jnp.float32),
            pltpu.VMEM_SHARED((NUM_R,), jnp.float32),
            pltpu.VMEM_SHARED((NUM_R,), jnp.float32),
            pltpu.SemaphoreType.DMA,
        ],
        compiler_params=pltpu.CompilerParams(
            needs_layout_passes=False, use_tc_tiling_on_sc=False),
    )(val, w, idx)


def _finish_body(d0_ref, d1_ref, a0_ref, a1_ref, mn_ref, mx_ref, fac_ref,
                 out_ref):
    depth = (d0_ref[...] + d1_ref[...]) / (a0_ref[...] + a1_ref[...] + 1e-10)
    mn = jnp.min(mn_ref[...])
    mx = jnp.max(mx_ref[...])
    out_ref[...] = jnp.clip(depth, mn, mx) * fac_ref[...]


def _tc_finish(d0, d1, a0, a1, mn, mx, fac):
    return pl.pallas_call(
        _finish_body,
        out_shape=jax.ShapeDtypeStruct((NUM_R // LANE, LANE), jnp.float32),
    )(d0, d1, a0, a1, mn, mx, fac)


def kernel(weights, starts, ends, factor_depth_coords, ray_indices, num_rays):
    del num_rays
    w2 = weights.reshape(ROWS, LANE)
    s2 = starts.reshape(ROWS, LANE)
    e2 = ends.reshape(ROWS, LANE)
    val, mn, mx = _tc_prep(w2, s2, e2)
    idx = ray_indices.astype(jnp.int32)
    tabs = _sc_scatter(val.reshape(NUM_S), weights.reshape(NUM_S), idx)
    tr = NUM_R // LANE
    d0 = tabs[0, 0].reshape(tr, LANE)
    d1 = tabs[1, 0].reshape(tr, LANE)
    a0 = tabs[0, 1].reshape(tr, LANE)
    a1 = tabs[1, 1].reshape(tr, LANE)
    fac = factor_depth_coords.reshape(tr, LANE)
    out = _tc_finish(d0, d1, a0, a1, mn, mx, fac)
    return out.reshape(NUM_R, 1)


# speedup vs baseline: 2.0834x; 2.0819x over previous
import jax
import jax.numpy as jnp
from jax import lax
from jax.experimental import pallas as pl
from jax.experimental.pallas import tpu as pltpu
from jax.experimental.pallas import tpu_sc as plsc

NUM_S = 4194304
NUM_R = 65536
NC = 2
NS = 16
NW = NC * NS
LANE = 128
ROWS = NUM_S // LANE

PREP_BLK = 1024
PREP_GRID = ROWS // PREP_BLK

SAMP_PER_W = NUM_S // NW
SAMP_PER_CHUNK = 16384
N_CHUNKS = SAMP_PER_W // SAMP_PER_CHUNK
WIN = 16384
FLUSH = 2048
UNROLL = 8


def _prep_body(w_ref, s_ref, e_ref, val_ref, mn_ref, mx_ref):
    steps = (s_ref[...] + e_ref[...]) * 0.5
    val_ref[...] = w_ref[...] * steps
    mn_ref[...] = jnp.full((1, 1, LANE), jnp.min(steps), jnp.float32)
    mx_ref[...] = jnp.full((1, 1, LANE), jnp.max(steps), jnp.float32)


def _tc_prep(w, s, e):
    blk = pl.BlockSpec((PREP_BLK, LANE), lambda g: (g, 0))
    row = pl.BlockSpec((1, 1, LANE), lambda g: (g, 0, 0))
    return pl.pallas_call(
        _prep_body,
        grid=(PREP_GRID,),
        in_specs=[blk, blk, blk],
        out_specs=[blk, row, row],
        out_shape=[
            jax.ShapeDtypeStruct((ROWS, LANE), jnp.float32),
            jax.ShapeDtypeStruct((PREP_GRID, 1, LANE), jnp.float32),
            jax.ShapeDtypeStruct((PREP_GRID, 1, LANE), jnp.float32),
        ],
    )(w, s, e)


def _sc_body(val_hbm, w_hbm, idx_hbm, tabs_hbm,
             valb, wb, idxb, idxs, wdtab, watab, fidxb, zb, dtab, atab, sem):
    c = lax.axis_index("c")
    s = lax.axis_index("s")
    wid = c * NS + s

    stripe = NUM_R // NS

    def _zero(i, _):
        zb[pl.ds(i * 16, 16)] = jnp.zeros((16,), jnp.float32)
        return 0

    lax.fori_loop(0, stripe // 16, _zero, 0)
    pltpu.sync_copy(zb, dtab.at[pl.ds(s * stripe, stripe)])
    pltpu.sync_copy(zb, atab.at[pl.ds(s * stripe, stripe)])

    def _zl(i, _):
        z = jnp.zeros((16,), jnp.float32)
        wdtab[pl.ds(i * 16, 16)] = z
        watab[pl.ds(i * 16, 16)] = z
        return 0

    lax.fori_loop(0, WIN // 16, _zl, 0)
    plsc.subcore_barrier()

    s0 = wid * SAMP_PER_W

    def _chunk(ck, carry):
        f, maxs = carry
        sb = s0 + ck * SAMP_PER_CHUNK
        pltpu.sync_copy(val_hbm.at[pl.ds(sb, SAMP_PER_CHUNK)], valb)
        pltpu.sync_copy(w_hbm.at[pl.ds(sb, SAMP_PER_CHUNK)], wb)
        pltpu.sync_copy(idx_hbm.at[pl.ds(sb, SAMP_PER_CHUNK)],
                        idxb.at[pl.ds(0, SAMP_PER_CHUNK)])
        f = lax.select(ck == 0, idxb[pl.ds(0, 16)][0], f)
        span = idxb[pl.ds(SAMP_PER_CHUNK - 16, 16)][15] - f + 1
        fast = span <= WIN

        def _fast():
            fv = jnp.full((16,), f, jnp.int32)
            lane = lax.iota(jnp.int32, 16)
            c15 = lane == 15
            cn15 = lane != 15

            def _vec(j, _):
                for k in range(UNROLL):
                    base = (j * UNROLL + k) * 16
                    idxv = idxb[pl.ds(base, 16)]
                    nxtv = idxb[pl.ds(base + 1, 16)]
                    csv = plsc.cumsum(valb[pl.ds(base, 16)])
                    csw = plsc.cumsum(wb[pl.ds(base, 16)])
                    cmp = idxv != nxtv
                    posm = cmp | c15
                    negm = cmp & cn15
                    ivp = idxv - fv
                    ivn = nxtv - fv
                    plsc.addupdate_scatter(wdtab, [ivp], csv, mask=posm)
                    plsc.addupdate_scatter(wdtab, [ivn], -csv, mask=negm)
                    plsc.addupdate_scatter(watab, [ivp], csw, mask=posm)
                    plsc.addupdate_scatter(watab, [ivn], -csw, mask=negm)
                return 0

            lax.fori_loop(0, SAMP_PER_CHUNK // (16 * UNROLL), _vec, 0)

        def _slow():
            pltpu.sync_copy(idx_hbm.at[pl.ds(sb, SAMP_PER_CHUNK)], idxs)
            d_cp = pltpu.async_copy(valb, dtab.at[idxs], sem, add=True)
            a_cp = pltpu.async_copy(wb, atab.at[idxs], sem, add=True)
            d_cp.wait()
            a_cp.wait()

        lax.cond(fast, _fast, _slow)
        maxs = lax.select(fast, jnp.maximum(maxs, span), maxs)
        return f, maxs

    f, maxs = lax.fori_loop(0, N_CHUNKS, _chunk,
                            (jnp.int32(0), jnp.int32(0)))

    nblk = (maxs + FLUSH - 1) // FLUSH
    lane16 = lax.iota(jnp.int32, 16)

    def _flush(b, _):
        base = f + b * FLUSH

        def _bld(j, _):
            fidxb[pl.ds(j * 16, 16)] = jnp.minimum(
                lane16 + (base + j * 16), NUM_R - 1)
            return 0

        lax.fori_loop(0, FLUSH // 16, _bld, 0)
        off = b * FLUSH
        d_cp = pltpu.async_copy(wdtab.at[pl.ds(off, FLUSH)],
                                dtab.at[fidxb], sem, add=True)
        a_cp = pltpu.async_copy(watab.at[pl.ds(off, FLUSH)],
                                atab.at[fidxb], sem, add=True)
        d_cp.wait()
        a_cp.wait()
        return 0

    lax.fori_loop(0, nblk, _flush, 0)
    plsc.subcore_barrier()

    @pl.when(s == 0)
    def _dump():
        pltpu.sync_copy(dtab, tabs_hbm.at[c, 0])
        pltpu.sync_copy(atab, tabs_hbm.at[c, 1])


def _sc_scatter(val, w, idx):
    mesh = plsc.VectorSubcoreMesh(core_axis_name="c", subcore_axis_name="s")
    return pl.kernel(
        _sc_body,
        out_type=jax.ShapeDtypeStruct((NC, 2, NUM_R), jnp.float32),
        mesh=mesh,
        scratch_types=[
            pltpu.VMEM((SAMP_PER_CHUNK,), jnp.float32),
            pltpu.VMEM((SAMP_PER_CHUNK,), jnp.float32),
            pltpu.VMEM((SAMP_PER_CHUNK + 16,), jnp.int32),
            pltpu.VMEM((SAMP_PER_CHUNK,), jnp.int32),
            pltpu.VMEM((WIN,), jnp.float32),
            pltpu.VMEM((WIN,), jnp.float32),
            pltpu.VMEM((FLUSH,), jnp.int32),
            pltpu.VMEM((NUM_R // NS,), jnp.float32),
            pltpu.VMEM_SHARED((NUM_R,), jnp.float32),
            pltpu.VMEM_SHARED((NUM_R,), jnp.float32),
            pltpu.SemaphoreType.DMA,
        ],
        compiler_params=pltpu.CompilerParams(
            needs_layout_passes=False, use_tc_tiling_on_sc=False),
    )(val, w, idx)


def _finish_body(d0_ref, d1_ref, a0_ref, a1_ref, mn_ref, mx_ref, fac_ref,
                 out_ref):
    depth = (d0_ref[...] + d1_ref[...]) / (a0_ref[...] + a1_ref[...] + 1e-10)
    mn = jnp.min(mn_ref[...])
    mx = jnp.max(mx_ref[...])
    out_ref[...] = jnp.clip(depth, mn, mx) * fac_ref[...]


def _tc_finish(d0, d1, a0, a1, mn, mx, fac):
    return pl.pallas_call(
        _finish_body,
        out_shape=jax.ShapeDtypeStruct((NUM_R // LANE, LANE), jnp.float32),
    )(d0, d1, a0, a1, mn, mx, fac)


def kernel(weights, starts, ends, factor_depth_coords, ray_indices, num_rays):
    del num_rays
    w2 = weights.reshape(ROWS, LANE)
    s2 = starts.reshape(ROWS, LANE)
    e2 = ends.reshape(ROWS, LANE)
    val, mn, mx = _tc_prep(w2, s2, e2)
    idx = ray_indices.astype(jnp.int32)
    tabs = _sc_scatter(val.reshape(NUM_S), weights.reshape(NUM_S), idx)
    tr = NUM_R // LANE
    d0 = tabs[0, 0].reshape(tr, LANE)
    d1 = tabs[1, 0].reshape(tr, LANE)
    a0 = tabs[0, 1].reshape(tr, LANE)
    a1 = tabs[1, 1].reshape(tr, LANE)
    fac = factor_depth_coords.reshape(tr, LANE)
    out = _tc_finish(d0, d1, a0, a1, mn, mx, fac)
    return out.reshape(NUM_R, 1)


# speedup vs baseline: 2.4029x; 1.1534x over previous
import jax
import jax.numpy as jnp
from jax import lax
from jax.experimental import pallas as pl
from jax.experimental.pallas import tpu as pltpu
from jax.experimental.pallas import tpu_sc as plsc

NUM_S = 4194304
NUM_R = 65536
NC = 2
NS = 16
NW = NC * NS
LANE = 128
ROWS = NUM_S // LANE

PREP_BLK = 1024
PREP_GRID = ROWS // PREP_BLK

SAMP_PER_W = NUM_S // NW
SAMP_PER_CHUNK = 8192
N_CHUNKS = SAMP_PER_W // SAMP_PER_CHUNK
WIN = 16384
FLUSH = 2048
UNROLL = 8


def _prep_body(w_ref, s_ref, e_ref, val_ref, mn_ref, mx_ref):
    steps = (s_ref[...] + e_ref[...]) * 0.5
    val_ref[...] = w_ref[...] * steps
    mn_ref[...] = jnp.full((1, 1, LANE), jnp.min(steps), jnp.float32)
    mx_ref[...] = jnp.full((1, 1, LANE), jnp.max(steps), jnp.float32)


def _tc_prep(w, s, e):
    blk = pl.BlockSpec((PREP_BLK, LANE), lambda g: (g, 0))
    row = pl.BlockSpec((1, 1, LANE), lambda g: (g, 0, 0))
    return pl.pallas_call(
        _prep_body,
        grid=(PREP_GRID,),
        in_specs=[blk, blk, blk],
        out_specs=[blk, row, row],
        out_shape=[
            jax.ShapeDtypeStruct((ROWS, LANE), jnp.float32),
            jax.ShapeDtypeStruct((PREP_GRID, 1, LANE), jnp.float32),
            jax.ShapeDtypeStruct((PREP_GRID, 1, LANE), jnp.float32),
        ],
    )(w, s, e)


def _sc_body(val_hbm, w_hbm, idx_hbm, tabs_hbm,
             valb0, wb0, idxb0, valb1, wb1, idxb1,
             idxs, wdtab, watab, fidxb, zb, dtab, atab, sem, lsem):
    c = lax.axis_index("c")
    s = lax.axis_index("s")
    wid = c * NS + s

    stripe = NUM_R // NS

    def _zero(i, _):
        zb[pl.ds(i * 16, 16)] = jnp.zeros((16,), jnp.float32)
        return 0

    lax.fori_loop(0, stripe // 16, _zero, 0)
    pltpu.sync_copy(zb, dtab.at[pl.ds(s * stripe, stripe)])
    pltpu.sync_copy(zb, atab.at[pl.ds(s * stripe, stripe)])

    def _zl(i, _):
        z = jnp.zeros((16,), jnp.float32)
        wdtab[pl.ds(i * 16, 16)] = z
        watab[pl.ds(i * 16, 16)] = z
        return 0

    lax.fori_loop(0, WIN // 16, _zl, 0)
    plsc.subcore_barrier()

    s0 = wid * SAMP_PER_W
    lane = lax.iota(jnp.int32, 16)
    c15 = lane == 15
    cn15 = lane != 15
    bufs = ((valb0, wb0, idxb0), (valb1, wb1, idxb1))

    def _load(ck, valb, wb, idxb):
        sb = s0 + ck * SAMP_PER_CHUNK
        return (
            pltpu.async_copy(val_hbm.at[pl.ds(sb, SAMP_PER_CHUNK)], valb,
                             lsem),
            pltpu.async_copy(w_hbm.at[pl.ds(sb, SAMP_PER_CHUNK)], wb, lsem),
            pltpu.async_copy(idx_hbm.at[pl.ds(sb, SAMP_PER_CHUNK)],
                             idxb.at[pl.ds(0, SAMP_PER_CHUNK)], lsem),
        )

    def _process(ck, valb, wb, idxb, f, maxs):
        sb = s0 + ck * SAMP_PER_CHUNK
        if ck == 0:
            f = idxb[pl.ds(0, 16)][0]
        span = idxb[pl.ds(SAMP_PER_CHUNK - 16, 16)][15] - f + 1
        fast = span <= WIN

        def _fast():
            fv = jnp.full((16,), f, jnp.int32)

            def _vec(j, _):
                for k in range(UNROLL):
                    base = (j * UNROLL + k) * 16
                    idxv = idxb[pl.ds(base, 16)]
                    nxtv = idxb[pl.ds(base + 1, 16)]
                    csv = plsc.cumsum(valb[pl.ds(base, 16)])
                    csw = plsc.cumsum(wb[pl.ds(base, 16)])
                    cmp = idxv != nxtv
                    posm = cmp | c15
                    negm = cmp & cn15
                    ivp = idxv - fv
                    ivn = nxtv - fv
                    plsc.addupdate_scatter(wdtab, [ivp], csv, mask=posm)
                    plsc.addupdate_scatter(wdtab, [ivn], -csv, mask=negm)
                    plsc.addupdate_scatter(watab, [ivp], csw, mask=posm)
                    plsc.addupdate_scatter(watab, [ivn], -csw, mask=negm)
                return 0

            lax.fori_loop(0, SAMP_PER_CHUNK // (16 * UNROLL), _vec, 0)

        def _slow():
            pltpu.sync_copy(idx_hbm.at[pl.ds(sb, SAMP_PER_CHUNK)], idxs)
            d_cp = pltpu.async_copy(valb, dtab.at[idxs], sem, add=True)
            a_cp = pltpu.async_copy(wb, atab.at[idxs], sem, add=True)
            d_cp.wait()
            a_cp.wait()

        lax.cond(fast, _fast, _slow)
        maxs = lax.select(fast, jnp.maximum(maxs, span), maxs)
        return f, maxs

    f = jnp.int32(0)
    maxs = jnp.int32(0)
    pending = _load(0, *bufs[0])
    for ck in range(N_CHUNKS):
        for cp in pending:
            cp.wait()
        if ck + 1 < N_CHUNKS:
            pending = _load(ck + 1, *bufs[(ck + 1) % 2])
        f, maxs = _process(ck, *bufs[ck % 2], f, maxs)

    nblk = (maxs + FLUSH - 1) // FLUSH
    lane16 = lax.iota(jnp.int32, 16)

    def _flush(b, _):
        base = f + b * FLUSH

        def _bld(j, _):
            fidxb[pl.ds(j * 16, 16)] = jnp.minimum(
                lane16 + (base + j * 16), NUM_R - 1)
            return 0

        lax.fori_loop(0, FLUSH // 16, _bld, 0)
        off = b * FLUSH
        d_cp = pltpu.async_copy(wdtab.at[pl.ds(off, FLUSH)],
                                dtab.at[fidxb], sem, add=True)
        a_cp = pltpu.async_copy(watab.at[pl.ds(off, FLUSH)],
                                atab.at[fidxb], sem, add=True)
        d_cp.wait()
        a_cp.wait()
        return 0

    lax.fori_loop(0, nblk, _flush, 0)
    plsc.subcore_barrier()

    @pl.when(s == 0)
    def _dump():
        pltpu.sync_copy(dtab, tabs_hbm.at[c, 0])
        pltpu.sync_copy(atab, tabs_hbm.at[c, 1])


def _sc_scatter(val, w, idx):
    mesh = plsc.VectorSubcoreMesh(core_axis_name="c", subcore_axis_name="s")
    return pl.kernel(
        _sc_body,
        out_type=jax.ShapeDtypeStruct((NC, 2, NUM_R), jnp.float32),
        mesh=mesh,
        scratch_types=[
            pltpu.VMEM((SAMP_PER_CHUNK,), jnp.float32),
            pltpu.VMEM((SAMP_PER_CHUNK,), jnp.float32),
            pltpu.VMEM((SAMP_PER_CHUNK + 16,), jnp.int32),
            pltpu.VMEM((SAMP_PER_CHUNK,), jnp.float32),
            pltpu.VMEM((SAMP_PER_CHUNK,), jnp.float32),
            pltpu.VMEM((SAMP_PER_CHUNK + 16,), jnp.int32),
            pltpu.VMEM((SAMP_PER_CHUNK,), jnp.int32),
            pltpu.VMEM((WIN,), jnp.float32),
            pltpu.VMEM((WIN,), jnp.float32),
            pltpu.VMEM((FLUSH,), jnp.int32),
            pltpu.VMEM((NUM_R // NS,), jnp.float32),
            pltpu.VMEM_SHARED((NUM_R,), jnp.float32),
            pltpu.VMEM_SHARED((NUM_R,), jnp.float32),
            pltpu.SemaphoreType.DMA,
            pltpu.SemaphoreType.DMA,
        ],
        compiler_params=pltpu.CompilerParams(
            needs_layout_passes=False, use_tc_tiling_on_sc=False),
    )(val, w, idx)


def _finish_body(d0_ref, d1_ref, a0_ref, a1_ref, mn_ref, mx_ref, fac_ref,
                 out_ref):
    depth = (d0_ref[...] + d1_ref[...]) / (a0_ref[...] + a1_ref[...] + 1e-10)
    mn = jnp.min(mn_ref[...])
    mx = jnp.max(mx_ref[...])
    out_ref[...] = jnp.clip(depth, mn, mx) * fac_ref[...]


def _tc_finish(d0, d1, a0, a1, mn, mx, fac):
    return pl.pallas_call(
        _finish_body,
        out_shape=jax.ShapeDtypeStruct((NUM_R // LANE, LANE), jnp.float32),
    )(d0, d1, a0, a1, mn, mx, fac)


def kernel(weights, starts, ends, factor_depth_coords, ray_indices, num_rays):
    del num_rays
    w2 = weights.reshape(ROWS, LANE)
    s2 = starts.reshape(ROWS, LANE)
    e2 = ends.reshape(ROWS, LANE)
    val, mn, mx = _tc_prep(w2, s2, e2)
    idx = ray_indices.astype(jnp.int32)
    tabs = _sc_scatter(val.reshape(NUM_S), weights.reshape(NUM_S), idx)
    tr = NUM_R // LANE
    d0 = tabs[0, 0].reshape(tr, LANE)
    d1 = tabs[1, 0].reshape(tr, LANE)
    a0 = tabs[0, 1].reshape(tr, LANE)
    a1 = tabs[1, 1].reshape(tr, LANE)
    fac = factor_depth_coords.reshape(tr, LANE)
    out = _tc_finish(d0, d1, a0, a1, mn, mx, fac)
    return out.reshape(NUM_R, 1)


# speedup vs baseline: 3.7685x; 1.5683x over previous
import jax
import jax.numpy as jnp
from jax import lax
from jax.experimental import pallas as pl
from jax.experimental.pallas import tpu as pltpu
from jax.experimental.pallas import tpu_sc as plsc

NUM_S = 4194304
NUM_R = 65536
NC = 2
NS = 16
NW = NC * NS
LANE = 128
ROWS = NUM_S // LANE

PREP_BLK = 1024
PREP_GRID = ROWS // PREP_BLK

SAMP_PER_W = NUM_S // NW
SAMP_PER_CHUNK = 8192
N_CHUNKS = SAMP_PER_W // SAMP_PER_CHUNK
WIN = 16384
FLUSH = 2048
UNROLL = 8


def _prep_body(w_ref, s_ref, e_ref, val_ref, mn_ref, mx_ref):
    steps = (s_ref[...] + e_ref[...]) * 0.5
    val_ref[...] = w_ref[...] * steps
    mn_ref[...] = jnp.full((1, 1, LANE), jnp.min(steps), jnp.float32)
    mx_ref[...] = jnp.full((1, 1, LANE), jnp.max(steps), jnp.float32)


def _tc_prep(w, s, e):
    blk = pl.BlockSpec((PREP_BLK, LANE), lambda g: (g, 0))
    row = pl.BlockSpec((1, 1, LANE), lambda g: (g, 0, 0))
    return pl.pallas_call(
        _prep_body,
        grid=(PREP_GRID,),
        in_specs=[blk, blk, blk],
        out_specs=[blk, row, row],
        out_shape=[
            jax.ShapeDtypeStruct((ROWS, LANE), jnp.float32),
            jax.ShapeDtypeStruct((PREP_GRID, 1, LANE), jnp.float32),
            jax.ShapeDtypeStruct((PREP_GRID, 1, LANE), jnp.float32),
        ],
    )(w, s, e)


def _sc_body(val_hbm, w_hbm, idx_hbm, tabs_hbm,
             valb0, wb0, idxb0, valb1, wb1, idxb1,
             idxs, wdtab, watab, fidxb, zb, dtab, atab, sem, lsem):
    c = lax.axis_index("c")
    s = lax.axis_index("s")
    wid = c * NS + s

    stripe = NUM_R // NS

    def _zero(i, _):
        zb[pl.ds(i * 16, 16)] = jnp.zeros((16,), jnp.float32)
        return 0

    lax.fori_loop(0, stripe // 16, _zero, 0)
    pltpu.sync_copy(zb, dtab.at[pl.ds(s * stripe, stripe)])
    pltpu.sync_copy(zb, atab.at[pl.ds(s * stripe, stripe)])

    def _zl(i, _):
        z = jnp.zeros((16,), jnp.float32)
        wdtab[pl.ds(i * 16, 16)] = z
        watab[pl.ds(i * 16, 16)] = z
        return 0

    lax.fori_loop(0, WIN // 16, _zl, 0)
    plsc.subcore_barrier()

    s0 = wid * SAMP_PER_W
    lane = lax.iota(jnp.int32, 16)
    c15 = lane == 15
    cn15 = lane != 15
    bufs = ((valb0, wb0, idxb0), (valb1, wb1, idxb1))

    def _load(ck, valb, wb, idxb):
        sb = s0 + ck * SAMP_PER_CHUNK
        return (
            pltpu.async_copy(val_hbm.at[pl.ds(sb, SAMP_PER_CHUNK)], valb,
                             lsem),
            pltpu.async_copy(w_hbm.at[pl.ds(sb, SAMP_PER_CHUNK)], wb, lsem),
            pltpu.async_copy(idx_hbm.at[pl.ds(sb, SAMP_PER_CHUNK)],
                             idxb.at[pl.ds(0, SAMP_PER_CHUNK)], lsem),
        )

    def _process(ck, valb, wb, idxb, f, maxs):
        sb = s0 + ck * SAMP_PER_CHUNK
        if ck == 0:
            f = idxb[pl.ds(0, 16)][0]
        span = idxb[pl.ds(SAMP_PER_CHUNK - 16, 16)][15] - f + 1
        fast = span <= WIN

        def _fast():
            fv = jnp.full((16,), f, jnp.int32)

            @plsc.parallel_loop(0, SAMP_PER_CHUNK // 16, unroll=UNROLL)
            def _vec(j):
                base = j * 16
                idxv = idxb[pl.ds(base, 16)]
                nxtv = idxb[pl.ds(base + 1, 16)]
                csv = plsc.cumsum(valb[pl.ds(base, 16)])
                csw = plsc.cumsum(wb[pl.ds(base, 16)])
                cmp = idxv != nxtv
                posm = cmp | c15
                negm = cmp & cn15
                ivp = idxv - fv
                ivn = nxtv - fv
                plsc.addupdate_scatter(wdtab, [ivp], csv, mask=posm)
                plsc.addupdate_scatter(wdtab, [ivn], -csv, mask=negm)
                plsc.addupdate_scatter(watab, [ivp], csw, mask=posm)
                plsc.addupdate_scatter(watab, [ivn], -csw, mask=negm)

        def _slow():
            pltpu.sync_copy(idx_hbm.at[pl.ds(sb, SAMP_PER_CHUNK)], idxs)
            d_cp = pltpu.async_copy(valb, dtab.at[idxs], sem, add=True)
            a_cp = pltpu.async_copy(wb, atab.at[idxs], sem, add=True)
            d_cp.wait()
            a_cp.wait()

        lax.cond(fast, _fast, _slow)
        maxs = lax.select(fast, jnp.maximum(maxs, span), maxs)
        return f, maxs

    f = jnp.int32(0)
    maxs = jnp.int32(0)
    pending = _load(0, *bufs[0])
    for ck in range(N_CHUNKS):
        for cp in pending:
            cp.wait()
        if ck + 1 < N_CHUNKS:
            pending = _load(ck + 1, *bufs[(ck + 1) % 2])
        f, maxs = _process(ck, *bufs[ck % 2], f, maxs)

    nblk = (maxs + FLUSH - 1) // FLUSH
    lane16 = lax.iota(jnp.int32, 16)

    def _flush(b, _):
        base = f + b * FLUSH

        def _bld(j, _):
            fidxb[pl.ds(j * 16, 16)] = jnp.minimum(
                lane16 + (base + j * 16), NUM_R - 1)
            return 0

        lax.fori_loop(0, FLUSH // 16, _bld, 0)
        off = b * FLUSH
        d_cp = pltpu.async_copy(wdtab.at[pl.ds(off, FLUSH)],
                                dtab.at[fidxb], sem, add=True)
        a_cp = pltpu.async_copy(watab.at[pl.ds(off, FLUSH)],
                                atab.at[fidxb], sem, add=True)
        d_cp.wait()
        a_cp.wait()
        return 0

    lax.fori_loop(0, nblk, _flush, 0)
    plsc.subcore_barrier()

    @pl.when(s == 0)
    def _dump():
        pltpu.sync_copy(dtab, tabs_hbm.at[c, 0])
        pltpu.sync_copy(atab, tabs_hbm.at[c, 1])


def _sc_scatter(val, w, idx):
    mesh = plsc.VectorSubcoreMesh(core_axis_name="c", subcore_axis_name="s")
    return pl.kernel(
        _sc_body,
        out_type=jax.ShapeDtypeStruct((NC, 2, NUM_R), jnp.float32),
        mesh=mesh,
        scratch_types=[
            pltpu.VMEM((SAMP_PER_CHUNK,), jnp.float32),
            pltpu.VMEM((SAMP_PER_CHUNK,), jnp.float32),
            pltpu.VMEM((SAMP_PER_CHUNK + 16,), jnp.int32),
            pltpu.VMEM((SAMP_PER_CHUNK,), jnp.float32),
            pltpu.VMEM((SAMP_PER_CHUNK,), jnp.float32),
            pltpu.VMEM((SAMP_PER_CHUNK + 16,), jnp.int32),
            pltpu.VMEM((SAMP_PER_CHUNK,), jnp.int32),
            pltpu.VMEM((WIN,), jnp.float32),
            pltpu.VMEM((WIN,), jnp.float32),
            pltpu.VMEM((FLUSH,), jnp.int32),
            pltpu.VMEM((NUM_R // NS,), jnp.float32),
            pltpu.VMEM_SHARED((NUM_R,), jnp.float32),
            pltpu.VMEM_SHARED((NUM_R,), jnp.float32),
            pltpu.SemaphoreType.DMA,
            pltpu.SemaphoreType.DMA,
        ],
        compiler_params=pltpu.CompilerParams(
            needs_layout_passes=False, use_tc_tiling_on_sc=False),
    )(val, w, idx)


def _finish_body(d0_ref, d1_ref, a0_ref, a1_ref, mn_ref, mx_ref, fac_ref,
                 out_ref):
    depth = (d0_ref[...] + d1_ref[...]) / (a0_ref[...] + a1_ref[...] + 1e-10)
    mn = jnp.min(mn_ref[...])
    mx = jnp.max(mx_ref[...])
    out_ref[...] = jnp.clip(depth, mn, mx) * fac_ref[...]


def _tc_finish(d0, d1, a0, a1, mn, mx, fac):
    return pl.pallas_call(
        _finish_body,
        out_shape=jax.ShapeDtypeStruct((NUM_R // LANE, LANE), jnp.float32),
    )(d0, d1, a0, a1, mn, mx, fac)


def kernel(weights, starts, ends, factor_depth_coords, ray_indices, num_rays):
    del num_rays
    w2 = weights.reshape(ROWS, LANE)
    s2 = starts.reshape(ROWS, LANE)
    e2 = ends.reshape(ROWS, LANE)
    val, mn, mx = _tc_prep(w2, s2, e2)
    idx = ray_indices.astype(jnp.int32)
    tabs = _sc_scatter(val.reshape(NUM_S), weights.reshape(NUM_S), idx)
    tr = NUM_R // LANE
    d0 = tabs[0, 0].reshape(tr, LANE)
    d1 = tabs[1, 0].reshape(tr, LANE)
    a0 = tabs[0, 1].reshape(tr, LANE)
    a1 = tabs[1, 1].reshape(tr, LANE)
    fac = factor_depth_coords.reshape(tr, LANE)
    out = _tc_finish(d0, d1, a0, a1, mn, mx, fac)
    return out.reshape(NUM_R, 1)


# speedup vs baseline: 4.4553x; 1.1822x over previous
import jax
import jax.numpy as jnp
from jax import lax
from jax.experimental import pallas as pl
from jax.experimental.pallas import tpu as pltpu
from jax.experimental.pallas import tpu_sc as plsc

NUM_S = 4194304
NUM_R = 65536
NC = 2
NS = 16
NW = NC * NS
LANE = 128

SAMP_PER_W = NUM_S // NW
SAMP_PER_CHUNK = 8192
N_CHUNKS = SAMP_PER_W // SAMP_PER_CHUNK
WIN = 16384
FLUSH = 2048
UNROLL = 8


def _sc_body(w_hbm, s_hbm, e_hbm, idx_hbm, tabs_hbm, mm_hbm,
             wb0, sb0, eb0, idxb0, wb1, sb1, eb1, idxb1,
             valt, idxs, wdtab, watab, fidxb, zb, mmb,
             dtab, atab, sem, lsem):
    c = lax.axis_index("c")
    s = lax.axis_index("s")
    wid = c * NS + s

    stripe = NUM_R // NS

    def _zero(i, _):
        zb[pl.ds(i * 16, 16)] = jnp.zeros((16,), jnp.float32)
        return 0

    lax.fori_loop(0, stripe // 16, _zero, 0)
    pltpu.sync_copy(zb, dtab.at[pl.ds(s * stripe, stripe)])
    pltpu.sync_copy(zb, atab.at[pl.ds(s * stripe, stripe)])

    def _zl(i, _):
        z = jnp.zeros((16,), jnp.float32)
        wdtab[pl.ds(i * 16, 16)] = z
        watab[pl.ds(i * 16, 16)] = z
        return 0

    lax.fori_loop(0, WIN // 16, _zl, 0)
    plsc.subcore_barrier()

    s0 = wid * SAMP_PER_W
    lane = lax.iota(jnp.int32, 16)
    c15 = lane == 15
    cn15 = lane != 15
    bufs = ((wb0, sb0, eb0, idxb0), (wb1, sb1, eb1, idxb1))

    def _load(ck, wb, sb_, eb, idxb):
        sb = s0 + ck * SAMP_PER_CHUNK
        sl = pl.ds(sb, SAMP_PER_CHUNK)
        return (
            pltpu.async_copy(w_hbm.at[sl], wb, lsem),
            pltpu.async_copy(s_hbm.at[sl], sb_, lsem),
            pltpu.async_copy(e_hbm.at[sl], eb, lsem),
            pltpu.async_copy(idx_hbm.at[sl], idxb.at[pl.ds(0, SAMP_PER_CHUNK)],
                             lsem),
        )

    def _process(ck, wb, sb_, eb, idxb, f, maxs, mnv, mxv):
        sb = s0 + ck * SAMP_PER_CHUNK
        if ck == 0:
            f = idxb[pl.ds(0, 16)][0]
        span = idxb[pl.ds(SAMP_PER_CHUNK - 16, 16)][15] - f + 1
        fast = span <= WIN

        def _fast():
            fv = jnp.full((16,), f, jnp.int32)

            @plsc.parallel_loop(0, SAMP_PER_CHUNK // 16, unroll=UNROLL,
                                carry=(mnv, mxv))
            def _vec(j, mm):
                mn, mx = mm
                base = j * 16
                idxv = idxb[pl.ds(base, 16)]
                nxtv = idxb[pl.ds(base + 1, 16)]
                stepv = (sb_[pl.ds(base, 16)] + eb[pl.ds(base, 16)]) * 0.5
                wv = wb[pl.ds(base, 16)]
                csv = plsc.cumsum(wv * stepv)
                csw = plsc.cumsum(wv)
                cmp = idxv != nxtv
                posm = cmp | c15
                negm = cmp & cn15
                ivp = idxv - fv
                ivn = nxtv - fv
                plsc.addupdate_scatter(wdtab, [ivp], csv, mask=posm)
                plsc.addupdate_scatter(wdtab, [ivn], -csv, mask=negm)
                plsc.addupdate_scatter(watab, [ivp], csw, mask=posm)
                plsc.addupdate_scatter(watab, [ivn], -csw, mask=negm)
                return (jnp.minimum(mn, stepv), jnp.maximum(mx, stepv))

            return _vec

        def _slow():
            @plsc.parallel_loop(0, SAMP_PER_CHUNK // 16, unroll=UNROLL,
                                carry=(mnv, mxv))
            def _vec(j, mm):
                mn, mx = mm
                base = j * 16
                stepv = (sb_[pl.ds(base, 16)] + eb[pl.ds(base, 16)]) * 0.5
                valt[pl.ds(base, 16)] = wb[pl.ds(base, 16)] * stepv
                return (jnp.minimum(mn, stepv), jnp.maximum(mx, stepv))

            pltpu.sync_copy(idx_hbm.at[pl.ds(sb, SAMP_PER_CHUNK)], idxs)
            d_cp = pltpu.async_copy(valt, dtab.at[idxs], sem, add=True)
            a_cp = pltpu.async_copy(wb, atab.at[idxs], sem, add=True)
            d_cp.wait()
            a_cp.wait()
            return _vec

        mnv, mxv = lax.cond(fast, _fast, _slow)
        maxs = lax.select(fast, jnp.maximum(maxs, span), maxs)
        return f, maxs, mnv, mxv

    f = jnp.int32(0)
    maxs = jnp.int32(0)
    mnv = jnp.full((16,), jnp.inf, jnp.float32)
    mxv = jnp.full((16,), -jnp.inf, jnp.float32)
    pending = _load(0, *bufs[0])
    for ck in range(N_CHUNKS):
        for cp in pending:
            cp.wait()
        if ck + 1 < N_CHUNKS:
            pending = _load(ck + 1, *bufs[(ck + 1) % 2])
        f, maxs, mnv, mxv = _process(ck, *bufs[ck % 2], f, maxs, mnv, mxv)

    mmb[pl.ds(0, 16)] = mnv
    mmb[pl.ds(16, 16)] = mxv
    pltpu.sync_copy(mmb.at[pl.ds(0, 16)], mm_hbm.at[0, wid])
    pltpu.sync_copy(mmb.at[pl.ds(16, 16)], mm_hbm.at[1, wid])

    nblk = (maxs + FLUSH - 1) // FLUSH

    def _flush(b, _):
        base = f + b * FLUSH

        def _bld(j, _):
            fidxb[pl.ds(j * 16, 16)] = jnp.minimum(
                lane + (base + j * 16), NUM_R - 1)
            return 0

        lax.fori_loop(0, FLUSH // 16, _bld, 0)
        off = b * FLUSH
        d_cp = pltpu.async_copy(wdtab.at[pl.ds(off, FLUSH)],
                                dtab.at[fidxb], sem, add=True)
        a_cp = pltpu.async_copy(watab.at[pl.ds(off, FLUSH)],
                                atab.at[fidxb], sem, add=True)
        d_cp.wait()
        a_cp.wait()
        return 0

    lax.fori_loop(0, nblk, _flush, 0)
    plsc.subcore_barrier()

    @pl.when(s == 0)
    def _dump():
        pltpu.sync_copy(dtab, tabs_hbm.at[c, 0])
        pltpu.sync_copy(atab, tabs_hbm.at[c, 1])


def _sc_scatter(w, s, e, idx):
    mesh = plsc.VectorSubcoreMesh(core_axis_name="c", subcore_axis_name="s")
    return pl.kernel(
        _sc_body,
        out_type=[
            jax.ShapeDtypeStruct((NC, 2, NUM_R), jnp.float32),
            jax.ShapeDtypeStruct((2, NW, 16), jnp.float32),
        ],
        mesh=mesh,
        scratch_types=[
            pltpu.VMEM((SAMP_PER_CHUNK,), jnp.float32),
            pltpu.VMEM((SAMP_PER_CHUNK,), jnp.float32),
            pltpu.VMEM((SAMP_PER_CHUNK,), jnp.float32),
            pltpu.VMEM((SAMP_PER_CHUNK + 16,), jnp.int32),
            pltpu.VMEM((SAMP_PER_CHUNK,), jnp.float32),
            pltpu.VMEM((SAMP_PER_CHUNK,), jnp.float32),
            pltpu.VMEM((SAMP_PER_CHUNK,), jnp.float32),
            pltpu.VMEM((SAMP_PER_CHUNK + 16,), jnp.int32),
            pltpu.VMEM((SAMP_PER_CHUNK,), jnp.float32),
            pltpu.VMEM((SAMP_PER_CHUNK,), jnp.int32),
            pltpu.VMEM((WIN,), jnp.float32),
            pltpu.VMEM((WIN,), jnp.float32),
            pltpu.VMEM((FLUSH,), jnp.int32),
            pltpu.VMEM((NUM_R // NS,), jnp.float32),
            pltpu.VMEM((32,), jnp.float32),
            pltpu.VMEM_SHARED((NUM_R,), jnp.float32),
            pltpu.VMEM_SHARED((NUM_R,), jnp.float32),
            pltpu.SemaphoreType.DMA,
            pltpu.SemaphoreType.DMA,
        ],
        compiler_params=pltpu.CompilerParams(
            needs_layout_passes=False, use_tc_tiling_on_sc=False),
    )(w, s, e, idx)


def _finish_body(d0_ref, d1_ref, a0_ref, a1_ref, mn_ref, mx_ref, fac_ref,
                 out_ref):
    depth = (d0_ref[...] + d1_ref[...]) / (a0_ref[...] + a1_ref[...] + 1e-10)
    mn = jnp.min(mn_ref[...])
    mx = jnp.max(mx_ref[...])
    out_ref[...] = jnp.clip(depth, mn, mx) * fac_ref[...]


def _tc_finish(d0, d1, a0, a1, mn, mx, fac):
    return pl.pallas_call(
        _finish_body,
        out_shape=jax.ShapeDtypeStruct((NUM_R // LANE, LANE), jnp.float32),
    )(d0, d1, a0, a1, mn, mx, fac)


def kernel(weights, starts, ends, factor_depth_coords, ray_indices, num_rays):
    del num_rays
    idx = ray_indices.astype(jnp.int32)
    tabs, mm = _sc_scatter(weights.reshape(NUM_S), starts.reshape(NUM_S),
                           ends.reshape(NUM_S), idx)
    tr = NUM_R // LANE
    d0 = tabs[0, 0].reshape(tr, LANE)
    d1 = tabs[1, 0].reshape(tr, LANE)
    a0 = tabs[0, 1].reshape(tr, LANE)
    a1 = tabs[1, 1].reshape(tr, LANE)
    mn = mm[0].reshape(4, LANE)
    mx = mm[1].reshape(4, LANE)
    fac = factor_depth_coords.reshape(tr, LANE)
    out = _tc_finish(d0, d1, a0, a1, mn, mx, fac)
    return out.reshape(NUM_R, 1)
